# baseline (speedup 1.0000x reference)
import numpy as np

import jax
import jax.numpy as jnp
from jax.experimental import pallas as pl
from jax.experimental.pallas import tpu as pltpu

D_MODEL = 1024
MAX_LEN = 2048
S_BLK = 128


def _pe_tables(max_len, d_model, s_blk):
    half = np.exp(np.arange(0, d_model, 2, dtype=np.float64) * (-np.log(10000.0) / d_model))
    d = np.repeat(half, 2)
    phi = np.zeros(d_model, dtype=np.float64)
    phi[1::2] = np.pi / 2.0
    g = np.arange(max_len // s_blk, dtype=np.float64)[:, None] * s_blk
    r = np.arange(s_blk, dtype=np.float64)[:, None]
    coarse_sin = np.sin(g * d).astype(np.float32)
    coarse_cos = np.cos(g * d).astype(np.float32)
    fine_sin = np.sin(r * d + phi).astype(np.float32)
    fine_cos = np.cos(r * d + phi).astype(np.float32)
    return coarse_sin, coarse_cos, fine_sin, fine_cos


def _body(x_ref, f_ref, tab_ref, cs_ref, cc_ref, fs_ref, fc_ref, o_ref):
    t0 = tab_ref[0, :]
    t1 = tab_ref[1, :]
    g = pl.program_id(0)
    pe = cs_ref[g, :] * fc_ref[...] + cc_ref[g, :] * fs_ref[...]
    base = pe + 0.3 * t0
    dv = 0.3 * (t1 - t0)
    f = jnp.clip(f_ref[...], 0, 1).astype(jnp.float32)
    for b in range(x_ref.shape[0]):
        o_ref[b, :, :] = x_ref[b, :, :] + base + f[b][:, None] * dv


def kernel(x, crisis_flags, crisis_table):
    B, S, D = x.shape
    flags = crisis_flags.astype(jnp.int32)
    cs, cc, fs, fc = _pe_tables(S, D, S_BLK)
    grid = (S // S_BLK,)
    return pl.pallas_call(
        _body,
        grid=grid,
        in_specs=[
            pl.BlockSpec((B, S_BLK, D), lambda g: (0, g, 0)),
            pl.BlockSpec((B, S_BLK), lambda g: (0, g)),
            pl.BlockSpec((2, D), lambda g: (0, 0)),
            pl.BlockSpec((S // S_BLK, D), lambda g: (0, 0)),
            pl.BlockSpec((S // S_BLK, D), lambda g: (0, 0)),
            pl.BlockSpec((S_BLK, D), lambda g: (0, 0)),
            pl.BlockSpec((S_BLK, D), lambda g: (0, 0)),
        ],
        out_specs=pl.BlockSpec((B, S_BLK, D), lambda g: (0, g, 0)),
        out_shape=jax.ShapeDtypeStruct((B, S, D), x.dtype),
        compiler_params=pltpu.CompilerParams(
            dimension_semantics=("parallel",),
        ),
    )(x, flags, crisis_table,
      jnp.asarray(cs), jnp.asarray(cc), jnp.asarray(fs), jnp.asarray(fc))


# speedup vs baseline: 1.1136x; 1.1136x over previous
import numpy as np

import jax
import jax.numpy as jnp
from jax.experimental import pallas as pl
from jax.experimental.pallas import tpu as pltpu

D_MODEL = 1024
MAX_LEN = 2048
S_BLK = 512


def _pe_tables(max_len, d_model, s_blk):
    half = np.exp(np.arange(0, d_model, 2, dtype=np.float64) * (-np.log(10000.0) / d_model))
    d = np.repeat(half, 2)
    phi = np.zeros(d_model, dtype=np.float64)
    phi[1::2] = np.pi / 2.0
    g = np.arange(max_len // s_blk, dtype=np.float64)[:, None] * s_blk
    r = np.arange(s_blk, dtype=np.float64)[:, None]
    coarse_sin = np.sin(g * d).astype(np.float32)
    coarse_cos = np.cos(g * d).astype(np.float32)
    fine_sin = np.sin(r * d + phi).astype(np.float32)
    fine_cos = np.cos(r * d + phi).astype(np.float32)
    return coarse_sin, coarse_cos, fine_sin, fine_cos


def _body(x_ref, f_ref, tab_ref, cs_ref, cc_ref, fs_ref, fc_ref, o_ref):
    t0 = tab_ref[0, :]
    t1 = tab_ref[1, :]
    g = pl.program_id(0)
    pe = cs_ref[g, :] * fc_ref[...] + cc_ref[g, :] * fs_ref[...]
    base = pe + 0.3 * t0
    dv = 0.3 * (t1 - t0)
    f = jnp.clip(f_ref[...], 0, 1).astype(jnp.float32)
    for b in range(x_ref.shape[0]):
        o_ref[b, :, :] = x_ref[b, :, :] + base + f[b][:, None] * dv


def kernel(x, crisis_flags, crisis_table):
    B, S, D = x.shape
    flags = crisis_flags.astype(jnp.int32)
    cs, cc, fs, fc = _pe_tables(S, D, S_BLK)
    grid = (S // S_BLK,)
    return pl.pallas_call(
        _body,
        grid=grid,
        in_specs=[
            pl.BlockSpec((B, S_BLK, D), lambda g: (0, g, 0)),
            pl.BlockSpec((B, S_BLK), lambda g: (0, g)),
            pl.BlockSpec((2, D), lambda g: (0, 0)),
            pl.BlockSpec((S // S_BLK, D), lambda g: (0, 0)),
            pl.BlockSpec((S // S_BLK, D), lambda g: (0, 0)),
            pl.BlockSpec((S_BLK, D), lambda g: (0, 0)),
            pl.BlockSpec((S_BLK, D), lambda g: (0, 0)),
        ],
        out_specs=pl.BlockSpec((B, S_BLK, D), lambda g: (0, g, 0)),
        out_shape=jax.ShapeDtypeStruct((B, S, D), x.dtype),
        compiler_params=pltpu.CompilerParams(
            dimension_semantics=("parallel",),
        ),
    )(x, flags, crisis_table,
      jnp.asarray(cs), jnp.asarray(cc), jnp.asarray(fs), jnp.asarray(fc))
